# baseline (speedup 1.0000x reference)
import functools

import jax
import jax.numpy as jnp
from jax import lax
from jax.experimental import pallas as pl
from jax.experimental.pallas import tpu as pltpu
from jax.experimental.pallas import tpu_sc as plsc

_NUM_CORES = 2
_NUM_SUBCORES = 16
_NW = _NUM_CORES * _NUM_SUBCORES
_VT = 1024
_REP = 64


def _sc_big_write(emb):
  mesh = plsc.VectorSubcoreMesh(
      core_axis_name="c", subcore_axis_name="s",
      num_cores=_NUM_CORES, num_subcores=_NUM_SUBCORES)

  @functools.partial(
      pl.kernel,
      out_type=jax.ShapeDtypeStruct((_NW, _REP, 1600, 16), jnp.float32),
      mesh=mesh,
      compiler_params=pltpu.CompilerParams(use_tc_tiling_on_sc=False),
      scratch_types=[
          pltpu.VMEM((1600, 16), jnp.float32),
          pltpu.SemaphoreType.DMA,
      ],
  )
  def big_write(emb_hbm, out_hbm, buf_v, sem):
    wid = lax.axis_index("s") * _NUM_CORES + lax.axis_index("c")
    pltpu.sync_copy(emb_hbm.at[pl.ds(0, 1600)], buf_v)
    def step(k, carry):
      cp1 = pltpu.async_copy(buf_v, out_hbm.at[wid, 2 * k], sem)
      cp2 = pltpu.async_copy(buf_v, out_hbm.at[wid, 2 * k + 1], sem)
      cp1.wait()
      cp2.wait()
      return carry
    lax.fori_loop(0, _REP // 2, step, 0)

  return big_write(emb)


def kernel(inputs, emb, W, b):
  B, C = inputs.shape
  V, D = emb.shape
  nvt = pl.cdiv(V, _VT)

  sc_junk = _sc_big_write(emb)

  def _zero_body(x_ref, o_ref):
    o_ref[...] = jnp.broadcast_to(x_ref[0, 0], o_ref.shape)

  tc_big = pl.pallas_call(
      _zero_body,
      grid=(nvt,),
      in_specs=[pl.BlockSpec((B, D), lambda j: (0, 0))],
      out_specs=pl.BlockSpec((B, _VT), lambda j: (0, j)),
      out_shape=jax.ShapeDtypeStruct((B, V), jnp.float32),
  )(W[:B])

  return tc_big, sc_junk


# speedup vs baseline: 1.7672x; 1.7672x over previous
import functools

import jax
import jax.numpy as jnp
from jax import lax
from jax.experimental import pallas as pl
from jax.experimental.pallas import tpu as pltpu
from jax.experimental.pallas import tpu_sc as plsc

_NUM_CORES = 2
_NUM_SUBCORES = 16
_NW = _NUM_CORES * _NUM_SUBCORES
_GCHUNK = 128

_VT = 1024
_NCHUNK = 4


def _gather_sum_sc(idx_flat, emb, B, C, D):
  per_w = B // _NW
  n_idx = per_w * C
  n_full = n_idx // _GCHUNK
  tail = n_idx - n_full * _GCHUNK

  mesh = plsc.VectorSubcoreMesh(
      core_axis_name="c", subcore_axis_name="s",
      num_cores=_NUM_CORES, num_subcores=_NUM_SUBCORES)

  @functools.partial(
      pl.kernel,
      out_type=jax.ShapeDtypeStruct((B, D), jnp.float32),
      mesh=mesh,
      compiler_params=pltpu.CompilerParams(use_tc_tiling_on_sc=False),
      scratch_types=[
          pltpu.VMEM((n_idx,), jnp.int32),
          pltpu.VMEM((n_idx, D), jnp.float32),
          pltpu.VMEM((per_w, D), jnp.float32),
          pltpu.SemaphoreType.DMA,
      ],
  )
  def gather_sum(emb_hbm, idx_hbm, out_hbm, idx_v, rows_v, acc_v, sem):
    wid = lax.axis_index("s") * _NUM_CORES + lax.axis_index("c")
    base = wid * n_idx
    pltpu.sync_copy(idx_hbm.at[pl.ds(base, n_idx)], idx_v)
    copies = []
    for j in range(n_full):
      copies.append(pltpu.async_copy(
          emb_hbm.at[idx_v.at[pl.ds(j * _GCHUNK, _GCHUNK)]],
          rows_v.at[pl.ds(j * _GCHUNK, _GCHUNK)], sem))
    if tail:
      copies.append(pltpu.async_copy(
          emb_hbm.at[idx_v.at[pl.ds(n_full * _GCHUNK, tail)]],
          rows_v.at[pl.ds(n_full * _GCHUNK, tail)], sem))
    for cp in copies:
      cp.wait()

    def row_body(r, carry):
      acc = rows_v[r * C]
      for c in range(1, C):
        acc = acc + rows_v[r * C + c]
      acc_v[r] = acc
      return carry

    lax.fori_loop(0, per_w, row_body, 0)
    pltpu.sync_copy(acc_v, out_hbm.at[pl.ds(wid * per_w, per_w)])

  return gather_sum(emb, idx_flat)


def _make_fused_body(CB):
  def fused_body(x_ref, w_ref, b_ref, o_ref, m2_ref, s2_ref):
    p = pl.program_id(0)
    j = pl.program_id(1)
    nchunk = pl.num_programs(0) - 1
    w = w_ref[...]
    bvec = b_ref[...]

    @pl.when(p < nchunk)
    def _stats():
      xs = x_ref[pl.ds(p * CB, CB), :]
      logits = lax.dot_general(
          xs, w, (((1,), (1,)), ((), ())),
          preferred_element_type=jnp.float32) + bvec
      tmax = jnp.max(logits, axis=1, keepdims=True)
      slot = lax.rem(p, 2)

      @pl.when(j == 0)
      def _():
        m2_ref[slot] = jnp.full((CB, 1), -jnp.inf, jnp.float32)
        s2_ref[slot] = jnp.zeros((CB, 1), jnp.float32)

      m_old = m2_ref[slot]
      m_new = jnp.maximum(m_old, tmax)
      s2_ref[slot] = (s2_ref[slot] * jnp.exp(m_old - m_new)
                      + jnp.sum(jnp.exp(logits - m_new), axis=1,
                                keepdims=True))
      m2_ref[slot] = m_new

    @pl.when(p > 0)
    def _write():
      q = p - 1
      slot = lax.rem(q, 2)
      xq = x_ref[pl.ds(q * CB, CB), :]
      logits = lax.dot_general(
          xq, w, (((1,), (1,)), ((), ())),
          preferred_element_type=jnp.float32) + bvec
      lse = m2_ref[slot] + jnp.log(s2_ref[slot])
      o_ref[...] = logits - lse

  return fused_body


def kernel(inputs, emb, W, b):
  B, C = inputs.shape
  V, D = emb.shape
  nvt = pl.cdiv(V, _VT)
  VP = nvt * _VT
  CB = B // _NCHUNK

  idx_flat = inputs.reshape(B * C).astype(jnp.int32)
  x = _gather_sum_sc(idx_flat, emb, B, C, D)

  W_pad = jnp.pad(W, ((0, VP - V), (0, 0)))
  b_pad = jnp.pad(b, (0, VP - V), constant_values=-1e30).reshape(1, VP)

  log_probs = pl.pallas_call(
      _make_fused_body(CB),
      grid=(_NCHUNK + 1, nvt),
      in_specs=[
          pl.BlockSpec((B, D), lambda p, j: (0, 0)),
          pl.BlockSpec((_VT, D), lambda p, j: (j, 0)),
          pl.BlockSpec((1, _VT), lambda p, j: (0, j)),
      ],
      out_specs=pl.BlockSpec(
          (CB, _VT),
          lambda p, j: (jnp.where(p > 0, p - 1, 0), jnp.where(p > 0, j, 0))),
      out_shape=jax.ShapeDtypeStruct((B, V), jnp.float32),
      scratch_shapes=[
          pltpu.VMEM((2, CB, 1), jnp.float32),
          pltpu.VMEM((2, CB, 1), jnp.float32),
      ],
  )(x, W_pad, b_pad)

  return log_probs
